# baseline (speedup 1.0000x reference)
import functools

import jax
import jax.numpy as jnp
from jax import lax
from jax.experimental import pallas as pl
from jax.experimental.pallas import tpu as pltpu
from jax.experimental.pallas import tpu_sc as plsc

_B, _H, _W, _C = 4, 224, 224, 192
_CP = 256
_N = 20000
_HW = _H * _W
_K = 32
_CPB = _N // _K
_NW = 32
_LANES = 16


def _interp_body(img_hbm, x_hbm, y_hbm, out_hbm,
                 xv, yv, idx4, wv4, gv, out_v, sems):
    wid = lax.axis_index("s") * 2 + lax.axis_index("c")
    lane = lax.iota(jnp.int32, _LANES)
    swz = [(lane + u) & (_LANES - 1) for u in range(_LANES)]

    def stage(b, c, s):
        base = c * _K
        pltpu.sync_copy(x_hbm.at[b, pl.ds(base, _K)], xv.at[s])
        pltpu.sync_copy(y_hbm.at[b, pl.ds(base, _K)], yv.at[s])
        for j in range(_K // _LANES):
            sl = pl.ds(j * _LANES, _LANES)
            xs = xv[s, sl]
            ys = yv[s, sl]
            x0i = xs.astype(jnp.int32)
            y0i = ys.astype(jnp.int32)
            x1i = jnp.minimum(x0i + 1, _W - 1)
            y1i = jnp.minimum(y0i + 1, _H - 1)
            x0f = x0i.astype(jnp.float32)
            x1f = x1i.astype(jnp.float32)
            y0f = y0i.astype(jnp.float32)
            y1f = y1i.astype(jnp.float32)
            ra = b * _HW + y0i * _W + x0i
            rb = b * _HW + y1i * _W + x0i
            idx4[s, 0, sl] = ra
            idx4[s, 1, sl] = rb
            idx4[s, 2, sl] = ra + (x1i - x0i)
            idx4[s, 3, sl] = rb + (x1i - x0i)
            wv4[s, 0, sl] = (x1f - xs) * (y1f - ys)
            wv4[s, 1, sl] = (x1f - xs) * (ys - y0f)
            wv4[s, 2, sl] = (xs - x0f) * (y1f - ys)
            wv4[s, 3, sl] = (xs - x0f) * (ys - y0f)
        for q in range(4):
            pltpu.async_copy(img_hbm.at[idx4.at[s, q]], gv.at[s, q], sems.at[s])

    def combine(b, c, s):
        for q in range(4):
            pltpu.make_async_copy(img_hbm.at[idx4.at[s, q]], gv.at[s, q],
                                  sems.at[s]).wait()
        for j in range(_K // _LANES):
            sl = pl.ds(j * _LANES, _LANES)
            rows = j * _LANES + lane
            wa = wv4[s, 0, sl]
            wb = wv4[s, 1, sl]
            wc = wv4[s, 2, sl]
            wd = wv4[s, 3, sl]

            def chan_body(ch, cols):
                for g in range(2):
                    for u in range(_LANES):
                        cu = cols + (g * _LANES) + swz[u]
                        va = plsc.load_gather(gv.at[s, 0], [rows, cu])
                        vb = plsc.load_gather(gv.at[s, 1], [rows, cu])
                        vc = plsc.load_gather(gv.at[s, 2], [rows, cu])
                        vd = plsc.load_gather(gv.at[s, 3], [rows, cu])
                        val = (wa * va + wb * vb) + (wc * vc + wd * vd)
                        plsc.store_scatter(out_v, [rows, cu], val)
                return cols + 2 * _LANES

            lax.fori_loop(0, _C // (2 * _LANES), chan_body,
                          jnp.zeros((_LANES,), jnp.int32))

        pltpu.sync_copy(out_v, out_hbm.at[b, pl.ds(c * _K, _K)])

    for b in range(_B):
        def at_idx(i):
            return i * _NW + wid

        @pl.when(at_idx(0) < _CPB)
        def _():
            stage(b, at_idx(0), 0)

        def pipe_body(it, carry):
            for par in range(2):
                i = 2 * it + par
                nxt = at_idx(i + 1)

                @pl.when(nxt < _CPB)
                def _():
                    stage(b, nxt, 1 - par)

                cur = at_idx(i)

                @pl.when(cur < _CPB)
                def _():
                    combine(b, cur, par)
            return carry

        niter = (_CPB + _NW - 1) // _NW
        lax.fori_loop(0, (niter + 1) // 2, pipe_body, 0)


@jax.jit
def _interp(imgs_pad, x, y):
    mesh = plsc.VectorSubcoreMesh(core_axis_name="c", subcore_axis_name="s")
    f = functools.partial(
        pl.kernel,
        mesh=mesh,
        compiler_params=pltpu.CompilerParams(needs_layout_passes=False),
        out_type=jax.ShapeDtypeStruct((_B, _N, _C), jnp.float32),
        scratch_types=[
            pltpu.VMEM((2, _K), jnp.float32),
            pltpu.VMEM((2, _K), jnp.float32),
            pltpu.VMEM((2, 4, _K), jnp.int32),
            pltpu.VMEM((2, 4, _K), jnp.float32),
            pltpu.VMEM((2, 4, _K, _CP), jnp.float32),
            pltpu.VMEM((_K, _C), jnp.float32),
            pltpu.SemaphoreType.DMA((2,)),
        ],
    )(_interp_body)
    return f(imgs_pad, x, y)


_HB = 56


def _pad_body(src_ref, dst_ref):
    dst_ref[:, :_C] = src_ref[...].reshape(_HB * _W, _C)
    dst_ref[:, _C:] = jnp.zeros((_HB * _W, _CP - _C), jnp.float32)


def _pad_tc(imgs):
    return pl.pallas_call(
        _pad_body,
        grid=(_B, _H // _HB),
        in_specs=[pl.BlockSpec((1, _HB, _W, _C), lambda b, h: (b, h, 0, 0))],
        out_specs=pl.BlockSpec((_HB * _W, _CP),
                               lambda b, h: (b * (_H // _HB) + h, 0)),
        out_shape=jax.ShapeDtypeStruct((_B * _HW, _CP), jnp.float32),
    )(imgs)


def kernel(imgs, x, y):
    return _interp(_pad_tc(imgs), x, y)


# speedup vs baseline: 1.0225x; 1.0225x over previous
import functools

import jax
import jax.numpy as jnp
from jax import lax
from jax.experimental import pallas as pl
from jax.experimental.pallas import tpu as pltpu
from jax.experimental.pallas import tpu_sc as plsc

_B, _H, _W, _C = 4, 224, 224, 192
_CP = 256
_N = 20000
_HW = _H * _W
_K = 32
_CPB = _N // _K
_NW = 32
_LANES = 16


def _make_interp_body(b):
    def _interp_body(img_hbm, x_hbm, y_hbm, out_hbm,
                     xv, yv, idx4, wv4, gv, out_v, sems):
        wid = lax.axis_index("s") * 2 + lax.axis_index("c")
        lane = lax.iota(jnp.int32, _LANES)
        swz = [(lane + u) & (_LANES - 1) for u in range(_LANES)]

        def stage(c, s):
            base = c * _K
            pltpu.sync_copy(x_hbm.at[b, pl.ds(base, _K)], xv.at[s])
            pltpu.sync_copy(y_hbm.at[b, pl.ds(base, _K)], yv.at[s])
            for j in range(_K // _LANES):
                sl = pl.ds(j * _LANES, _LANES)
                xs = xv[s, sl]
                ys = yv[s, sl]
                x0i = xs.astype(jnp.int32)
                y0i = ys.astype(jnp.int32)
                x1i = jnp.minimum(x0i + 1, _W - 1)
                y1i = jnp.minimum(y0i + 1, _H - 1)
                x0f = x0i.astype(jnp.float32)
                x1f = x1i.astype(jnp.float32)
                y0f = y0i.astype(jnp.float32)
                y1f = y1i.astype(jnp.float32)
                ra = y0i * _W + x0i
                rb = y1i * _W + x0i
                idx4[s, 0, sl] = ra
                idx4[s, 1, sl] = rb
                idx4[s, 2, sl] = ra + (x1i - x0i)
                idx4[s, 3, sl] = rb + (x1i - x0i)
                wv4[s, 0, sl] = (x1f - xs) * (y1f - ys)
                wv4[s, 1, sl] = (x1f - xs) * (ys - y0f)
                wv4[s, 2, sl] = (xs - x0f) * (y1f - ys)
                wv4[s, 3, sl] = (xs - x0f) * (ys - y0f)
            for q in range(4):
                pltpu.async_copy(img_hbm.at[idx4.at[s, q]], gv.at[s, q],
                                 sems.at[s])

        def combine(c, s):
            for q in range(4):
                pltpu.make_async_copy(img_hbm.at[idx4.at[s, q]], gv.at[s, q],
                                      sems.at[s]).wait()
            for j in range(_K // _LANES):
                sl = pl.ds(j * _LANES, _LANES)
                rows = j * _LANES + lane
                wa = wv4[s, 0, sl]
                wb = wv4[s, 1, sl]
                wc = wv4[s, 2, sl]
                wd = wv4[s, 3, sl]

                def chan_body(ch, cols):
                    for g in range(2):
                        for u in range(_LANES):
                            cu = cols + (g * _LANES) + swz[u]
                            va = plsc.load_gather(gv.at[s, 0], [rows, cu])
                            vb = plsc.load_gather(gv.at[s, 1], [rows, cu])
                            vc = plsc.load_gather(gv.at[s, 2], [rows, cu])
                            vd = plsc.load_gather(gv.at[s, 3], [rows, cu])
                            val = (wa * va + wb * vb) + (wc * vc + wd * vd)
                            plsc.store_scatter(out_v, [rows, cu], val)
                    return cols + 2 * _LANES

                lax.fori_loop(0, _C // (2 * _LANES), chan_body,
                              jnp.zeros((_LANES,), jnp.int32))

            pltpu.sync_copy(out_v, out_hbm.at[pl.ds(c * _K, _K)])

        def at_idx(i):
            return i * _NW + wid

        @pl.when(at_idx(0) < _CPB)
        def _():
            stage(at_idx(0), 0)

        def pipe_body(it, carry):
            for par in range(2):
                i = 2 * it + par
                nxt = at_idx(i + 1)

                @pl.when(nxt < _CPB)
                def _():
                    stage(nxt, 1 - par)

                cur = at_idx(i)

                @pl.when(cur < _CPB)
                def _():
                    combine(cur, par)
            return carry

        niter = (_CPB + _NW - 1) // _NW
        lax.fori_loop(0, (niter + 1) // 2, pipe_body, 0)

    return _interp_body


def _interp_one(b, imgs_pad_b, x, y):
    mesh = plsc.VectorSubcoreMesh(core_axis_name="c", subcore_axis_name="s")
    f = functools.partial(
        pl.kernel,
        mesh=mesh,
        compiler_params=pltpu.CompilerParams(needs_layout_passes=False),
        out_type=jax.ShapeDtypeStruct((_N, _C), jnp.float32),
        scratch_types=[
            pltpu.VMEM((2, _K), jnp.float32),
            pltpu.VMEM((2, _K), jnp.float32),
            pltpu.VMEM((2, 4, _K), jnp.int32),
            pltpu.VMEM((2, 4, _K), jnp.float32),
            pltpu.VMEM((2, 4, _K, _CP), jnp.float32),
            pltpu.VMEM((_K, _C), jnp.float32),
            pltpu.SemaphoreType.DMA((2,)),
        ],
    )(_make_interp_body(b))
    return f(imgs_pad_b, x, y)


_HB = 56


def _pad_body(src_ref, dst_ref):
    dst_ref[:, :_C] = src_ref[...].reshape(_HB * _W, _C)


def _pad_one(b, imgs):
    return pl.pallas_call(
        _pad_body,
        grid=(_H // _HB,),
        in_specs=[pl.BlockSpec((1, _HB, _W, _C), lambda h: (b, h, 0, 0))],
        out_specs=pl.BlockSpec((_HB * _W, _CP), lambda h: (h, 0)),
        out_shape=jax.ShapeDtypeStruct((_HW, _CP), jnp.float32),
    )(imgs)


_RB = 2000


def _place_body(prev_ref, piece_ref, out_ref):
    del prev_ref
    out_ref[...] = piece_ref[...].reshape(1, _RB, _C)


def _place(b, acc, piece):
    return pl.pallas_call(
        _place_body,
        grid=(_N // _RB,),
        in_specs=[pl.BlockSpec(memory_space=pltpu.MemorySpace.HBM),
                  pl.BlockSpec((_RB, _C), lambda r: (r, 0))],
        out_specs=pl.BlockSpec((1, _RB, _C), lambda r: (b, r, 0)),
        out_shape=jax.ShapeDtypeStruct((_B, _N, _C), jnp.float32),
        input_output_aliases={0: 0},
    )(acc, piece)


@jax.jit
def _run(imgs, x, y):
    pieces = []
    for b in range(_B):
        pieces.append(_interp_one(b, _pad_one(b, imgs), x, y))
    acc = jnp.zeros((_B, _N, _C), jnp.float32)
    for b in range(_B):
        acc = _place(b, acc, pieces[b])
    return acc


def kernel(imgs, x, y):
    return _run(imgs, x, y)


# speedup vs baseline: 1.0274x; 1.0048x over previous
import functools

import jax
import jax.numpy as jnp
from jax import lax
from jax.experimental import pallas as pl
from jax.experimental.pallas import tpu as pltpu
from jax.experimental.pallas import tpu_sc as plsc

_B, _H, _W, _C = 4, 224, 224, 192
_CP = 256
_N = 20000
_HW = _H * _W
_K = 32
_CPB = _N // _K
_NW = 32
_LANES = 16


def _make_interp_body(b):
    def _interp_body(img_hbm, x_hbm, y_hbm, out_hbm,
                     xv, yv, idx4, wv4, gv, out_v, sems):
        wid = lax.axis_index("s") * 2 + lax.axis_index("c")
        lane = lax.iota(jnp.int32, _LANES)
        swz = [(lane + u) & (_LANES - 1) for u in range(_LANES)]

        def stage(c, s):
            base = c * _K
            pltpu.sync_copy(x_hbm.at[b, pl.ds(base, _K)], xv.at[s])
            pltpu.sync_copy(y_hbm.at[b, pl.ds(base, _K)], yv.at[s])
            for j in range(_K // _LANES):
                sl = pl.ds(j * _LANES, _LANES)
                xs = xv[s, sl]
                ys = yv[s, sl]
                x0i = xs.astype(jnp.int32)
                y0i = ys.astype(jnp.int32)
                x1i = jnp.minimum(x0i + 1, _W - 1)
                y1i = jnp.minimum(y0i + 1, _H - 1)
                x0f = x0i.astype(jnp.float32)
                x1f = x1i.astype(jnp.float32)
                y0f = y0i.astype(jnp.float32)
                y1f = y1i.astype(jnp.float32)
                ra = y0i * _W + x0i
                rb = y1i * _W + x0i
                idx4[s, 0, sl] = ra
                idx4[s, 1, sl] = rb
                idx4[s, 2, sl] = ra + (x1i - x0i)
                idx4[s, 3, sl] = rb + (x1i - x0i)
                wv4[s, 0, sl] = (x1f - xs) * (y1f - ys)
                wv4[s, 1, sl] = (x1f - xs) * (ys - y0f)
                wv4[s, 2, sl] = (xs - x0f) * (y1f - ys)
                wv4[s, 3, sl] = (xs - x0f) * (ys - y0f)
            for q in range(4):
                pltpu.async_copy(img_hbm.at[idx4.at[s, q]], gv.at[s, q],
                                 sems.at[s])

        def combine(c, s):
            for q in range(4):
                pltpu.make_async_copy(img_hbm.at[idx4.at[s, q]], gv.at[s, q],
                                      sems.at[s]).wait()
            for j in range(_K // _LANES):
                sl = pl.ds(j * _LANES, _LANES)
                rows = j * _LANES + lane
                wa = wv4[s, 0, sl]
                wb = wv4[s, 1, sl]
                wc = wv4[s, 2, sl]
                wd = wv4[s, 3, sl]

                def chan_body(ch, cols):
                    for g in range(2):
                        for u in range(_LANES):
                            cu = cols + (g * _LANES) + swz[u]
                            va = plsc.load_gather(gv.at[s, 0], [rows, cu])
                            vb = plsc.load_gather(gv.at[s, 1], [rows, cu])
                            vc = plsc.load_gather(gv.at[s, 2], [rows, cu])
                            vd = plsc.load_gather(gv.at[s, 3], [rows, cu])
                            val = (wa * va + wb * vb) + (wc * vc + wd * vd)
                            plsc.store_scatter(out_v, [rows, cu], val)
                    return cols + 2 * _LANES

                lax.fori_loop(0, _C // (2 * _LANES), chan_body,
                              jnp.zeros((_LANES,), jnp.int32))

            pltpu.sync_copy(out_v, out_hbm.at[pl.ds(c * _K, _K)])

        def at_idx(i):
            return i * _NW + wid

        @pl.when(at_idx(0) < _CPB)
        def _():
            stage(at_idx(0), 0)

        def pipe_body(it, carry):
            for par in range(2):
                i = 2 * it + par
                nxt = at_idx(i + 1)

                @pl.when(nxt < _CPB)
                def _():
                    stage(nxt, 1 - par)

                cur = at_idx(i)

                @pl.when(cur < _CPB)
                def _():
                    combine(cur, par)
            return carry

        niter = (_CPB + _NW - 1) // _NW
        lax.fori_loop(0, (niter + 1) // 2, pipe_body, 0)

    return _interp_body


def _interp_one(b, imgs_pad_b, x, y):
    mesh = plsc.VectorSubcoreMesh(core_axis_name="c", subcore_axis_name="s")
    f = functools.partial(
        pl.kernel,
        mesh=mesh,
        compiler_params=pltpu.CompilerParams(needs_layout_passes=False),
        out_type=jax.ShapeDtypeStruct((_N, _C), jnp.float32),
        scratch_types=[
            pltpu.VMEM((2, _K), jnp.float32),
            pltpu.VMEM((2, _K), jnp.float32),
            pltpu.VMEM((2, 4, _K), jnp.int32),
            pltpu.VMEM((2, 4, _K), jnp.float32),
            pltpu.VMEM((2, 4, _K, _CP), jnp.float32),
            pltpu.VMEM((_K, _C), jnp.float32),
            pltpu.SemaphoreType.DMA((2,)),
        ],
    )(_make_interp_body(b))
    return f(imgs_pad_b, x, y)


_HB = 56


def _pad_body(src_ref, dst_ref):
    dst_ref[:, :_C] = src_ref[...].reshape(_HB * _W, _C)


def _pad_one(b, imgs):
    return pl.pallas_call(
        _pad_body,
        grid=(_H // _HB,),
        in_specs=[pl.BlockSpec((1, _HB, _W, _C), lambda h: (b, h, 0, 0))],
        out_specs=pl.BlockSpec((_HB * _W, _CP), lambda h: (h, 0)),
        out_shape=jax.ShapeDtypeStruct((_HW, _CP), jnp.float32),
    )(imgs)


_RB = 2000


def _place_body(prev_ref, piece_ref, out_ref):
    del prev_ref
    out_ref[...] = piece_ref[...].reshape(1, _RB, _C)


def _place_first_body(piece_ref, out_ref):
    out_ref[...] = piece_ref[...].reshape(1, _RB, _C)


def _place(b, acc, piece):
    if b == 0:
        return pl.pallas_call(
            _place_first_body,
            grid=(_N // _RB,),
            in_specs=[pl.BlockSpec((_RB, _C), lambda r: (r, 0))],
            out_specs=pl.BlockSpec((1, _RB, _C), lambda r: (0, r, 0)),
            out_shape=jax.ShapeDtypeStruct((_B, _N, _C), jnp.float32),
        )(piece)
    return pl.pallas_call(
        _place_body,
        grid=(_N // _RB,),
        in_specs=[pl.BlockSpec(memory_space=pltpu.MemorySpace.HBM),
                  pl.BlockSpec((_RB, _C), lambda r: (r, 0))],
        out_specs=pl.BlockSpec((1, _RB, _C), lambda r: (b, r, 0)),
        out_shape=jax.ShapeDtypeStruct((_B, _N, _C), jnp.float32),
        input_output_aliases={0: 0},
    )(acc, piece)


@jax.jit
def _run(imgs, x, y):
    pieces = []
    for b in range(_B):
        pieces.append(_interp_one(b, _pad_one(b, imgs), x, y))
    acc = None
    for b in range(_B):
        acc = _place(b, acc, pieces[b])
    return acc


def kernel(imgs, x, y):
    return _run(imgs, x, y)
